# baseline (speedup 1.0000x reference)
import functools

import jax
import jax.numpy as jnp
from jax import lax
from jax.experimental import pallas as pl
from jax.experimental.pallas import tpu as pltpu
from jax.experimental.pallas import tpu_sc as plsc

B, C, N, NPOINT = 8, 128, 100000, 16384
NC, NS, L = 2, 16, 16
NW = NC * NS
WPB = NW // B
CPW = C // WPB
NV = NPOINT // L

SEG = 40576
SEG2 = N // 128 * 128 - 2 * SEG
TAIL = N - N // 128 * 128
SPAN2 = SEG2 + TAIL
BASES = (0, SEG, 2 * SEG)
SPANS = (SEG, SEG, SPAN2)
ENC_CAP = NPOINT + 3 * L


def _gather_kernel(feat_hbm, tails_hbm, idx_hbm, out_hbm,
                   idx_v, enc_v, out_v, buf0_v, buf1_v, tail_v,
                   sem0, sem1, tsem, osem):
    wid = lax.axis_index("s") * NC + lax.axis_index("c")
    b = wid // WPB
    c0 = (wid % WPB) * CPW

    pltpu.sync_copy(idx_hbm.at[b], idx_v)

    j_iota = lax.iota(jnp.int32, L)
    bases_dyn = []
    trips = []
    off = jnp.int32(0)
    for k in range(3):
        base_k = off
        lo_v = jnp.full((L,), BASES[k], jnp.int32)
        span_u = jnp.full((L,), SPANS[k], jnp.uint32)

        def bbody(i, off, lo_v=lo_v, span_u=span_u):
            iv = idx_v[pl.ds(i * L, L)]
            loc = iv - lo_v
            inb = plsc.bitcast(loc, jnp.uint32) < span_u
            enc = ((j_iota + i * L) << 16) | (loc & 0xFFFF)
            plsc.store_compressed(enc_v.at[pl.ds(off, L)], enc, mask=inb)
            cnt = plsc.all_reduce_population_count(inb)
            return off + cnt[0]

        off = lax.fori_loop(0, NV, bbody, off)
        enc_v[pl.ds(off, L)] = jnp.full((L,), NPOINT << 16, jnp.int32)
        cnt_k = off - base_k
        trips_k = (cnt_k + (L - 1)) // L
        bases_dyn.append(base_k)
        trips.append(trips_k)
        off = base_k + trips_k * L

    def gather_pass(buf, k):
        base_k = bases_dyn[k]

        def body(t, _):
            enc = enc_v[pl.ds(base_k + t * L, L)]
            lidx = enc & 0xFFFF
            jv = lax.shift_right_logical(enc, 16)
            g = plsc.load_gather(buf, [lidx])
            plsc.store_scatter(out_v, [jv], g)
            return 0

        lax.fori_loop(0, trips[k], body, 0)

    bufs = (buf0_v, buf1_v)
    sems = (sem0, sem1)
    h0 = pltpu.make_async_copy(
        feat_hbm.at[b, c0, pl.ds(0, SEG)], bufs[0], sems[0])
    h0.start()
    h1 = pltpu.make_async_copy(
        feat_hbm.at[b, c0, pl.ds(SEG, SEG)], bufs[1], sems[1])
    h1.start()
    hout = None
    for r in range(CPW):
        p = r % 2
        q = 1 - p
        ht = pltpu.make_async_copy(
            tails_hbm.at[b, pl.ds((c0 + r) * TAIL, TAIL)], tail_v, tsem)
        ht.start()
        h0.wait()
        if hout is not None:
            hout.wait()
            hout = None
        gather_pass(bufs[p], 0)
        if r + 1 < CPW:
            h1_next = pltpu.make_async_copy(
                feat_hbm.at[b, c0 + r + 1, pl.ds(SEG, SEG)],
                bufs[p], sems[p])
            h1_next.start()
        h1.wait()
        gather_pass(bufs[q], 1)
        h2 = pltpu.make_async_copy(
            feat_hbm.at[b, c0 + r, pl.ds(2 * SEG, SEG2)],
            bufs[q].at[pl.ds(0, SEG2)], sems[q])
        h2.start()
        h2.wait()
        ht.wait()
        bufs[q][pl.ds(SEG2, L)] = tail_v[pl.ds(0, L)]
        bufs[q][pl.ds(SEG2 + L, L)] = tail_v[pl.ds(L, L)]
        gather_pass(bufs[q], 2)
        hout = pltpu.make_async_copy(
            out_v.at[pl.ds(0, NPOINT)], out_hbm.at[b, c0 + r], osem)
        hout.start()
        if r + 1 < CPW:
            h0 = pltpu.make_async_copy(
                feat_hbm.at[b, c0 + r + 1, pl.ds(0, SEG)],
                bufs[q].at[pl.ds(0, SEG)], sems[q])
            h0.start()
            h1 = h1_next
    if hout is not None:
        hout.wait()


@jax.jit
def kernel(features, idx):
    mesh = plsc.VectorSubcoreMesh(core_axis_name="c", subcore_axis_name="s")
    tails = features[:, :, N - TAIL:].reshape(B, C * TAIL)
    run = functools.partial(
        pl.kernel,
        mesh=mesh,
        compiler_params=pltpu.CompilerParams(needs_layout_passes=False),
        out_type=jax.ShapeDtypeStruct((B, C, NPOINT), jnp.float32),
        scratch_types=[
            pltpu.VMEM((NPOINT,), jnp.int32),
            pltpu.VMEM((ENC_CAP,), jnp.int32),
            pltpu.VMEM((NPOINT + L,), jnp.float32),
            pltpu.VMEM((SEG,), jnp.float32),
            pltpu.VMEM((SEG,), jnp.float32),
            pltpu.VMEM((TAIL,), jnp.float32),
            pltpu.SemaphoreType.DMA,
            pltpu.SemaphoreType.DMA,
            pltpu.SemaphoreType.DMA,
            pltpu.SemaphoreType.DMA,
        ],
    )(_gather_kernel)
    return run(features, tails, idx)


# speedup vs baseline: 1.4056x; 1.4056x over previous
import functools

import jax
import jax.numpy as jnp
from jax import lax
from jax.experimental import pallas as pl
from jax.experimental.pallas import tpu as pltpu
from jax.experimental.pallas import tpu_sc as plsc

B, C, N, NPOINT = 8, 128, 100000, 16384
NC, NS, L = 2, 16, 16
NW = NC * NS
WPB = NW // B
CPW = C // WPB
NV = NPOINT // L

SEG = 40576
SEG2 = N // 128 * 128 - 2 * SEG
TAIL = N - N // 128 * 128
SPAN2 = SEG2 + TAIL
BASES = (0, SEG, 2 * SEG)
SPANS = (SEG, SEG, SPAN2)
ENC_CAP = NPOINT + 3 * L


def _gather_kernel(feat_hbm, tails_hbm, idx_hbm, out_hbm,
                   idx_v, enc_v, out_v, buf0_v, buf1_v, tail_v,
                   sem0, sem1, tsem, osem):
    wid = lax.axis_index("s") * NC + lax.axis_index("c")
    b = wid // WPB
    c0 = (wid % WPB) * CPW

    pltpu.sync_copy(idx_hbm.at[b], idx_v)

    j_iota = lax.iota(jnp.int32, L)
    bases_dyn = []
    trips = []
    off = jnp.int32(0)
    for k in range(3):
        base_k = off
        lo_v = jnp.full((L,), BASES[k], jnp.int32)
        span_u = jnp.full((L,), SPANS[k], jnp.uint32)

        def bbody(i, off, lo_v=lo_v, span_u=span_u):
            iv = idx_v[pl.ds(i * L, L)]
            loc = iv - lo_v
            inb = plsc.bitcast(loc, jnp.uint32) < span_u
            enc = ((j_iota + i * L) << 16) | (loc & 0xFFFF)
            plsc.store_compressed(enc_v.at[pl.ds(off, L)], enc, mask=inb)
            cnt = plsc.all_reduce_population_count(inb)
            return off + cnt[0]

        off = lax.fori_loop(0, NV, bbody, off)
        enc_v[pl.ds(off, L)] = jnp.full((L,), NPOINT << 16, jnp.int32)
        cnt_k = off - base_k
        trips_k = (cnt_k + (L - 1)) // L
        bases_dyn.append(base_k)
        trips.append(trips_k)
        off = base_k + trips_k * L

    def gather_pass(buf, k):
        base_k = bases_dyn[k]

        @plsc.parallel_loop(0, trips[k], step=1, unroll=8)
        def _(t):
            enc = enc_v[pl.ds(base_k + t * L, L)]
            lidx = enc & 0xFFFF
            jv = lax.shift_right_logical(enc, 16)
            g = plsc.load_gather(buf, [lidx])
            plsc.store_scatter(out_v, [jv], g)

    bufs = (buf0_v, buf1_v)
    sems = (sem0, sem1)
    h0 = pltpu.make_async_copy(
        feat_hbm.at[b, c0, pl.ds(0, SEG)], bufs[0], sems[0])
    h0.start()
    h1 = pltpu.make_async_copy(
        feat_hbm.at[b, c0, pl.ds(SEG, SEG)], bufs[1], sems[1])
    h1.start()
    hout = None
    for r in range(CPW):
        p = r % 2
        q = 1 - p
        ht = pltpu.make_async_copy(
            tails_hbm.at[b, pl.ds((c0 + r) * TAIL, TAIL)], tail_v, tsem)
        ht.start()
        h0.wait()
        if hout is not None:
            hout.wait()
            hout = None
        gather_pass(bufs[p], 0)
        if r + 1 < CPW:
            h1_next = pltpu.make_async_copy(
                feat_hbm.at[b, c0 + r + 1, pl.ds(SEG, SEG)],
                bufs[p], sems[p])
            h1_next.start()
        h1.wait()
        gather_pass(bufs[q], 1)
        h2 = pltpu.make_async_copy(
            feat_hbm.at[b, c0 + r, pl.ds(2 * SEG, SEG2)],
            bufs[q].at[pl.ds(0, SEG2)], sems[q])
        h2.start()
        h2.wait()
        ht.wait()
        bufs[q][pl.ds(SEG2, L)] = tail_v[pl.ds(0, L)]
        bufs[q][pl.ds(SEG2 + L, L)] = tail_v[pl.ds(L, L)]
        gather_pass(bufs[q], 2)
        hout = pltpu.make_async_copy(
            out_v.at[pl.ds(0, NPOINT)], out_hbm.at[b, c0 + r], osem)
        hout.start()
        if r + 1 < CPW:
            h0 = pltpu.make_async_copy(
                feat_hbm.at[b, c0 + r + 1, pl.ds(0, SEG)],
                bufs[q].at[pl.ds(0, SEG)], sems[q])
            h0.start()
            h1 = h1_next
    if hout is not None:
        hout.wait()


@jax.jit
def kernel(features, idx):
    mesh = plsc.VectorSubcoreMesh(core_axis_name="c", subcore_axis_name="s")
    tails = features[:, :, N - TAIL:].reshape(B, C * TAIL)
    run = functools.partial(
        pl.kernel,
        mesh=mesh,
        compiler_params=pltpu.CompilerParams(needs_layout_passes=False),
        out_type=jax.ShapeDtypeStruct((B, C, NPOINT), jnp.float32),
        scratch_types=[
            pltpu.VMEM((NPOINT,), jnp.int32),
            pltpu.VMEM((ENC_CAP,), jnp.int32),
            pltpu.VMEM((NPOINT + L,), jnp.float32),
            pltpu.VMEM((SEG,), jnp.float32),
            pltpu.VMEM((SEG,), jnp.float32),
            pltpu.VMEM((TAIL,), jnp.float32),
            pltpu.SemaphoreType.DMA,
            pltpu.SemaphoreType.DMA,
            pltpu.SemaphoreType.DMA,
            pltpu.SemaphoreType.DMA,
        ],
    )(_gather_kernel)
    return run(features, tails, idx)


# speedup vs baseline: 1.4597x; 1.0385x over previous
import functools

import jax
import jax.numpy as jnp
from jax import lax
from jax.experimental import pallas as pl
from jax.experimental.pallas import tpu as pltpu
from jax.experimental.pallas import tpu_sc as plsc

B, C, N, NPOINT = 8, 128, 100000, 16384
NC, NS, L = 2, 16, 16
NW = NC * NS
WPB = NW // B
CPW = C // WPB
CHUNK = 4096
NCHUNK = NPOINT // CHUNK
VPC = CHUNK // L


def _gather_kernel(feat_hbm, idx_hbm, out_hbm, idx_v, row_v, obuf_v,
                   sem0, sem1, row_sem):
    wid = lax.axis_index("s") * NC + lax.axis_index("c")
    b = wid // WPB
    c0 = (wid % WPB) * CPW

    pltpu.sync_copy(idx_hbm.at[b], idx_v)

    sems = (sem0, sem1)
    pending = [None, None]
    for ci in range(CPW):
        c = c0 + ci
        cp = pltpu.make_async_copy(feat_hbm.at[b, c], row_v, row_sem)
        cp.start()
        cp.wait()
        for t in range(NCHUNK):
            sl = t % 2
            if pending[sl] is not None:
                pending[sl].wait()
                pending[sl] = None

            @plsc.parallel_loop(0, VPC, step=1, unroll=8)
            def body(jl, t=t, sl=sl):
                iv = idx_v[pl.ds(t * CHUNK + jl * L, L)]
                g = plsc.load_gather(row_v, [iv])
                obuf_v[sl, pl.ds(jl * L, L)] = g
            cp = pltpu.make_async_copy(
                obuf_v.at[sl], out_hbm.at[b, c, pl.ds(t * CHUNK, CHUNK)],
                sems[sl])
            cp.start()
            pending[sl] = cp
    for sl in range(2):
        if pending[sl] is not None:
            pending[sl].wait()


@jax.jit
def kernel(features, idx):
    mesh = plsc.VectorSubcoreMesh(core_axis_name="c", subcore_axis_name="s")
    run = functools.partial(
        pl.kernel,
        mesh=mesh,
        compiler_params=pltpu.CompilerParams(needs_layout_passes=False),
        out_type=jax.ShapeDtypeStruct((B, C, NPOINT), jnp.float32),
        scratch_types=[
            pltpu.VMEM((NPOINT,), jnp.int32),
            pltpu.VMEM((N,), jnp.float32),
            pltpu.VMEM((2, CHUNK), jnp.float32),
            pltpu.SemaphoreType.DMA,
            pltpu.SemaphoreType.DMA,
            pltpu.SemaphoreType.DMA,
        ],
    )(_gather_kernel)
    return run(features, idx)


# speedup vs baseline: 1.4980x; 1.0262x over previous
import functools

import jax
import jax.numpy as jnp
from jax import lax
from jax.experimental import pallas as pl
from jax.experimental.pallas import tpu as pltpu
from jax.experimental.pallas import tpu_sc as plsc

B, C, N, NPOINT = 8, 128, 100000, 16384
NC, NS, L = 2, 16, 16
NW = NC * NS
WPB = NW // B
CPW = C // WPB
NV = NPOINT // L

SEG = 40576
SEG2 = N // 128 * 128 - 2 * SEG
TAIL = N - N // 128 * 128
SPAN2 = SEG2 + TAIL
BASES = (0, SEG, 2 * SEG)
SPANS = (SEG, SEG, SPAN2)
ENC_CAP = NPOINT + 3 * L


def _gather_kernel(feat_hbm, tails_hbm, idx_hbm, out_hbm,
                   idx_v, enc_v, out_v, buf0_v, buf1_v, tail_v,
                   sem0, sem1, tsem, osem):
    wid = lax.axis_index("s") * NC + lax.axis_index("c")
    b = wid // WPB
    c0 = (wid % WPB) * CPW

    pltpu.sync_copy(idx_hbm.at[b], idx_v)

    j_iota = lax.iota(jnp.int32, L)
    bases_dyn = []
    trips = []
    off = jnp.int32(0)
    for k in range(3):
        base_k = off
        lo_v = jnp.full((L,), BASES[k], jnp.int32)
        span_u = jnp.full((L,), SPANS[k], jnp.uint32)

        def bbody(i, off, lo_v=lo_v, span_u=span_u):
            iv = idx_v[pl.ds(i * L, L)]
            loc = iv - lo_v
            inb = plsc.bitcast(loc, jnp.uint32) < span_u
            enc = ((j_iota + i * L) << 16) | (loc & 0xFFFF)
            plsc.store_compressed(enc_v.at[pl.ds(off, L)], enc, mask=inb)
            cnt = plsc.all_reduce_population_count(inb)
            return off + cnt[0]

        off = lax.fori_loop(0, NV, bbody, off)
        enc_v[pl.ds(off, L)] = jnp.full((L,), NPOINT << 16, jnp.int32)
        cnt_k = off - base_k
        trips_k = (cnt_k + (L - 1)) // L
        bases_dyn.append(base_k)
        trips.append(trips_k)
        off = base_k + trips_k * L

    def gather_pass(buf, k):
        base_k = bases_dyn[k]

        @plsc.parallel_loop(0, trips[k], step=1, unroll=8)
        def _(t):
            enc = enc_v[pl.ds(base_k + t * L, L)]
            lidx = enc & 0xFFFF
            jv = lax.shift_right_logical(enc, 16)
            g = plsc.load_gather(buf, [lidx])
            plsc.store_scatter(out_v, [jv], g)

    bufs = (buf0_v, buf1_v)
    sems = (sem0, sem1)
    h0 = pltpu.make_async_copy(
        feat_hbm.at[b, c0, pl.ds(0, SEG)], bufs[0], sems[0])
    h0.start()
    h1 = pltpu.make_async_copy(
        feat_hbm.at[b, c0, pl.ds(SEG, SEG)], bufs[1], sems[1])
    h1.start()
    hout = None
    for r in range(CPW):
        p = r % 2
        q = 1 - p
        ht = pltpu.make_async_copy(
            tails_hbm.at[b, pl.ds((c0 + r) * TAIL, TAIL)], tail_v, tsem)
        ht.start()
        h1.wait()
        if hout is not None:
            hout.wait()
            hout = None
        gather_pass(bufs[q], 1)
        h2 = pltpu.make_async_copy(
            feat_hbm.at[b, c0 + r, pl.ds(2 * SEG, SEG2)],
            bufs[q].at[pl.ds(0, SEG2)], sems[q])
        h2.start()
        h0.wait()
        gather_pass(bufs[p], 0)
        if r + 1 < CPW:
            h1_next = pltpu.make_async_copy(
                feat_hbm.at[b, c0 + r + 1, pl.ds(SEG, SEG)],
                bufs[p], sems[p])
            h1_next.start()
        h2.wait()
        ht.wait()
        bufs[q][pl.ds(SEG2, L)] = tail_v[pl.ds(0, L)]
        bufs[q][pl.ds(SEG2 + L, L)] = tail_v[pl.ds(L, L)]
        gather_pass(bufs[q], 2)
        hout = pltpu.make_async_copy(
            out_v.at[pl.ds(0, NPOINT)], out_hbm.at[b, c0 + r], osem)
        hout.start()
        if r + 1 < CPW:
            h0 = pltpu.make_async_copy(
                feat_hbm.at[b, c0 + r + 1, pl.ds(0, SEG)],
                bufs[q].at[pl.ds(0, SEG)], sems[q])
            h0.start()
            h1 = h1_next
    if hout is not None:
        hout.wait()


@jax.jit
def kernel(features, idx):
    mesh = plsc.VectorSubcoreMesh(core_axis_name="c", subcore_axis_name="s")
    tails = features[:, :, N - TAIL:].reshape(B, C * TAIL)
    run = functools.partial(
        pl.kernel,
        mesh=mesh,
        compiler_params=pltpu.CompilerParams(needs_layout_passes=False),
        out_type=jax.ShapeDtypeStruct((B, C, NPOINT), jnp.float32),
        scratch_types=[
            pltpu.VMEM((NPOINT,), jnp.int32),
            pltpu.VMEM((ENC_CAP,), jnp.int32),
            pltpu.VMEM((NPOINT + L,), jnp.float32),
            pltpu.VMEM((SEG,), jnp.float32),
            pltpu.VMEM((SEG,), jnp.float32),
            pltpu.VMEM((TAIL,), jnp.float32),
            pltpu.SemaphoreType.DMA,
            pltpu.SemaphoreType.DMA,
            pltpu.SemaphoreType.DMA,
            pltpu.SemaphoreType.DMA,
        ],
    )(_gather_kernel)
    return run(features, tails, idx)


# speedup vs baseline: 1.5014x; 1.0022x over previous
import functools

import jax
import jax.numpy as jnp
from jax import lax
from jax.experimental import pallas as pl
from jax.experimental.pallas import tpu as pltpu
from jax.experimental.pallas import tpu_sc as plsc

B, C, N, NPOINT = 8, 128, 100000, 16384
NC, NS, L = 2, 16, 16
NW = NC * NS
WPB = NW // B
CPW = C // WPB
NV = NPOINT // L

SEG = 40576
SEG2 = N // 128 * 128 - 2 * SEG
TAIL = N - N // 128 * 128
SPAN2 = SEG2 + TAIL
BASES = (0, SEG, 2 * SEG)
SPANS = (SEG, SEG, SPAN2)
ENC_CAP = NPOINT + 3 * L


def _gather_kernel(feat_hbm, tails_hbm, idx_hbm, out_hbm,
                   idx_v, enc_v, out_v, buf0_v, buf1_v, tail_v,
                   sem0, sem1, tsem, osem):
    wid = lax.axis_index("s") * NC + lax.axis_index("c")
    b = wid // WPB
    c0 = (wid % WPB) * CPW

    pltpu.sync_copy(idx_hbm.at[b], idx_v)

    j_iota = lax.iota(jnp.int32, L)
    bases_dyn = []
    trips = []
    off = jnp.int32(0)
    for k in range(3):
        base_k = off
        lo_v = jnp.full((L,), BASES[k], jnp.int32)
        span_u = jnp.full((L,), SPANS[k], jnp.uint32)

        def bbody(i, off, lo_v=lo_v, span_u=span_u):
            iv = idx_v[pl.ds(i * L, L)]
            loc = iv - lo_v
            inb = plsc.bitcast(loc, jnp.uint32) < span_u
            enc = ((j_iota + i * L) << 16) | (loc & 0xFFFF)
            plsc.store_compressed(enc_v.at[pl.ds(off, L)], enc, mask=inb)
            cnt = plsc.all_reduce_population_count(inb)
            return off + cnt[0]

        off = lax.fori_loop(0, NV, bbody, off, unroll=4)
        enc_v[pl.ds(off, L)] = jnp.full((L,), NPOINT << 16, jnp.int32)
        cnt_k = off - base_k
        trips_k = (cnt_k + (L - 1)) // L
        bases_dyn.append(base_k)
        trips.append(trips_k)
        off = base_k + trips_k * L

    def gather_pass(buf, k):
        base_k = bases_dyn[k]

        @plsc.parallel_loop(0, trips[k], step=1, unroll=8)
        def _(t):
            enc = enc_v[pl.ds(base_k + t * L, L)]
            lidx = enc & 0xFFFF
            jv = lax.shift_right_logical(enc, 16)
            g = plsc.load_gather(buf, [lidx])
            plsc.store_scatter(out_v, [jv], g)

    bufs = (buf0_v, buf1_v)
    sems = (sem0, sem1)
    h0 = pltpu.make_async_copy(
        feat_hbm.at[b, c0, pl.ds(0, SEG)], bufs[0], sems[0])
    h0.start()
    h1 = pltpu.make_async_copy(
        feat_hbm.at[b, c0, pl.ds(SEG, SEG)], bufs[1], sems[1])
    h1.start()
    hout = None
    for r in range(CPW):
        p = r % 2
        q = 1 - p
        ht = pltpu.make_async_copy(
            tails_hbm.at[b, pl.ds((c0 + r) * TAIL, TAIL)], tail_v, tsem)
        ht.start()
        h1.wait()
        if hout is not None:
            hout.wait()
            hout = None
        gather_pass(bufs[q], 1)
        h2 = pltpu.make_async_copy(
            feat_hbm.at[b, c0 + r, pl.ds(2 * SEG, SEG2)],
            bufs[q].at[pl.ds(0, SEG2)], sems[q])
        h2.start()
        h0.wait()
        gather_pass(bufs[p], 0)
        if r + 1 < CPW:
            h1_next = pltpu.make_async_copy(
                feat_hbm.at[b, c0 + r + 1, pl.ds(SEG, SEG)],
                bufs[p], sems[p])
            h1_next.start()
        h2.wait()
        ht.wait()
        bufs[q][pl.ds(SEG2, L)] = tail_v[pl.ds(0, L)]
        bufs[q][pl.ds(SEG2 + L, L)] = tail_v[pl.ds(L, L)]
        gather_pass(bufs[q], 2)
        hout = pltpu.make_async_copy(
            out_v.at[pl.ds(0, NPOINT)], out_hbm.at[b, c0 + r], osem)
        hout.start()
        if r + 1 < CPW:
            h0 = pltpu.make_async_copy(
                feat_hbm.at[b, c0 + r + 1, pl.ds(0, SEG)],
                bufs[q].at[pl.ds(0, SEG)], sems[q])
            h0.start()
            h1 = h1_next
    if hout is not None:
        hout.wait()


@jax.jit
def kernel(features, idx):
    mesh = plsc.VectorSubcoreMesh(core_axis_name="c", subcore_axis_name="s")
    tails = features[:, :, N - TAIL:].reshape(B, C * TAIL)
    run = functools.partial(
        pl.kernel,
        mesh=mesh,
        compiler_params=pltpu.CompilerParams(needs_layout_passes=False),
        out_type=jax.ShapeDtypeStruct((B, C, NPOINT), jnp.float32),
        scratch_types=[
            pltpu.VMEM((NPOINT,), jnp.int32),
            pltpu.VMEM((ENC_CAP,), jnp.int32),
            pltpu.VMEM((NPOINT + L,), jnp.float32),
            pltpu.VMEM((SEG,), jnp.float32),
            pltpu.VMEM((SEG,), jnp.float32),
            pltpu.VMEM((TAIL,), jnp.float32),
            pltpu.SemaphoreType.DMA,
            pltpu.SemaphoreType.DMA,
            pltpu.SemaphoreType.DMA,
            pltpu.SemaphoreType.DMA,
        ],
    )(_gather_kernel)
    return run(features, tails, idx)
